# baseline (device time: 54128 ns/iter reference)
import jax
import jax.numpy as jnp
from jax import lax
from jax.experimental import pallas as pl
from jax.experimental.pallas import tpu as pltpu

N_DEV = 4
N_TOK = 1024
D_IN = 512
D_OUT = 1024
E_LOCAL = 4
CAP = 51
ROWS = N_TOK // N_DEV


def kernel(x, router_W, route_idx, expert_W):
    del router_W

    def body(x_ref, ridx_ref, ew_ref, out_ref,
             partial_ref, accum_ref, comm_ref, send_sems, recv_sems):
        my = lax.axis_index("i")
        left = (my - 1) % N_DEV
        right = (my + 1) % N_DEV

        ridx = ridx_ref[:, :]
        local_ids = my * E_LOCAL + lax.broadcasted_iota(
            jnp.int32, (1, E_LOCAL), 1
        )
        onehot = (ridx == local_ids).astype(jnp.float32)
        cum = onehot
        sh = 1
        while sh < N_TOK:
            cum = cum + jnp.concatenate(
                [jnp.zeros((sh, E_LOCAL), jnp.float32), cum[:-sh]], axis=0
            )
            sh *= 2
        keep = onehot * (cum <= CAP).astype(jnp.float32)

        x_val = x_ref[:, :]
        acc = jnp.dot(x_val * keep[:, 0:1], ew_ref[0],
                      preferred_element_type=jnp.float32)
        for j in range(1, E_LOCAL):
            acc = acc + jnp.dot(x_val * keep[:, j:j + 1], ew_ref[j],
                                preferred_element_type=jnp.float32)
        partial_ref[:, :] = acc

        barrier = pltpu.get_barrier_semaphore()
        for nbr in [left, right]:
            pl.semaphore_signal(
                barrier, inc=1,
                device_id=(nbr,), device_id_type=pl.DeviceIdType.MESH,
            )
        pl.semaphore_wait(barrier, 2)

        for s in range(N_DEV - 1):
            c_send = (my - s - 1) % N_DEV
            if s == 0:
                src = partial_ref.at[pl.ds(c_send * ROWS, ROWS), :]
            else:
                src = accum_ref.at[s - 1]
            rdma = pltpu.make_async_remote_copy(
                src_ref=src,
                dst_ref=comm_ref.at[s],
                send_sem=send_sems.at[s],
                recv_sem=recv_sems.at[s],
                device_id=(right,),
                device_id_type=pl.DeviceIdType.MESH,
            )
            rdma.start()
            rdma.wait()

            c_recv = (my - s - 2) % N_DEV
            if s < N_DEV - 2:
                accum_ref[s] = (
                    comm_ref[s] + partial_ref[pl.ds(c_recv * ROWS, ROWS), :]
                )
            else:
                out_ref[:, :] = (
                    comm_ref[s] + partial_ref[pl.ds(my * ROWS, ROWS), :]
                )

    return pl.pallas_call(
        body,
        out_shape=jax.ShapeDtypeStruct((ROWS, D_OUT), jnp.float32),
        in_specs=[
            pl.BlockSpec(memory_space=pltpu.VMEM),
            pl.BlockSpec(memory_space=pltpu.VMEM),
            pl.BlockSpec(memory_space=pltpu.VMEM),
        ],
        out_specs=pl.BlockSpec(memory_space=pltpu.VMEM),
        scratch_shapes=[
            pltpu.VMEM((N_TOK, D_OUT), jnp.float32),
            pltpu.VMEM((N_DEV - 2, ROWS, D_OUT), jnp.float32),
            pltpu.VMEM((N_DEV - 1, ROWS, D_OUT), jnp.float32),
            pltpu.SemaphoreType.DMA((N_DEV - 1,)),
            pltpu.SemaphoreType.DMA((N_DEV - 1,)),
        ],
        compiler_params=pltpu.CompilerParams(collective_id=0),
    )(x, route_idx, expert_W)


# device time: 35335 ns/iter; 1.5319x vs baseline; 1.5319x over previous
import jax
import jax.numpy as jnp
from jax import lax
from jax.experimental import pallas as pl
from jax.experimental.pallas import tpu as pltpu

N_DEV = 4
N_TOK = 1024
D_IN = 512
D_OUT = 1024
E_LOCAL = 4
CAP = 51
ROWS = N_TOK // N_DEV
HALF = ROWS // 2


def kernel(x, router_W, route_idx, expert_W):
    del router_W

    def body(x_ref, ridx_ref, ew_ref, out_ref,
             keep_ref, partial_ref, accR_ref, accL_ref,
             commR_ref, commL_ref, sendR, recvR, sendL, recvL):
        my = lax.axis_index("i")
        left = (my - 1) % N_DEV
        right = (my + 1) % N_DEV

        ridx = ridx_ref[:, :]
        local_ids = my * E_LOCAL + lax.broadcasted_iota(
            jnp.int32, (1, E_LOCAL), 1
        )
        onehot = (ridx == local_ids).astype(jnp.float32)
        cum = onehot
        sh = 1
        while sh < N_TOK:
            cum = cum + jnp.concatenate(
                [jnp.zeros((sh, E_LOCAL), jnp.float32), cum[:-sh]], axis=0
            )
            sh *= 2
        keep_ref[:, :] = onehot * (cum <= CAP).astype(jnp.float32)

        def compute_chunk(c):
            off = c * ROWS
            xc = x_ref[pl.ds(off, ROWS), :]
            kc = keep_ref[pl.ds(off, ROWS), :]
            acc = jnp.dot(xc * kc[:, 0:1], ew_ref[0],
                          preferred_element_type=jnp.float32)
            for j in range(1, E_LOCAL):
                acc = acc + jnp.dot(xc * kc[:, j:j + 1], ew_ref[j],
                                    preferred_element_type=jnp.float32)
            partial_ref[pl.ds(off, ROWS), :] = acc

        compute_chunk((my - 1) % N_DEV)
        compute_chunk((my + 1) % N_DEV)

        barrier = pltpu.get_barrier_semaphore()
        for nbr in [left, right]:
            pl.semaphore_signal(
                barrier, inc=1,
                device_id=(nbr,), device_id_type=pl.DeviceIdType.MESH,
            )
        pl.semaphore_wait(barrier, 2)

        for s in range(N_DEV - 1):
            cR_send = (my - s - 1) % N_DEV
            cL_send = (my + s + 1) % N_DEV
            if s == 0:
                srcR = partial_ref.at[pl.ds(cR_send * ROWS, HALF), :]
                srcL = partial_ref.at[pl.ds(cL_send * ROWS + HALF, HALF), :]
            else:
                srcR = accR_ref.at[s - 1]
                srcL = accL_ref.at[s - 1]
            rdmaR = pltpu.make_async_remote_copy(
                src_ref=srcR, dst_ref=commR_ref.at[s],
                send_sem=sendR.at[s], recv_sem=recvR.at[s],
                device_id=(right,), device_id_type=pl.DeviceIdType.MESH,
            )
            rdmaL = pltpu.make_async_remote_copy(
                src_ref=srcL, dst_ref=commL_ref.at[s],
                send_sem=sendL.at[s], recv_sem=recvL.at[s],
                device_id=(left,), device_id_type=pl.DeviceIdType.MESH,
            )
            rdmaR.start()
            rdmaL.start()

            if s == 0:
                compute_chunk((my + 2) % N_DEV)
            elif s == 1:
                compute_chunk(my)

            rdmaR.wait()
            rdmaL.wait()

            cR_recv = (my - s - 2) % N_DEV
            cL_recv = (my + s + 2) % N_DEV
            if s < N_DEV - 2:
                accR_ref[s] = (
                    commR_ref[s]
                    + partial_ref[pl.ds(cR_recv * ROWS, HALF), :]
                )
                accL_ref[s] = (
                    commL_ref[s]
                    + partial_ref[pl.ds(cL_recv * ROWS + HALF, HALF), :]
                )
            else:
                out_ref[pl.ds(0, HALF), :] = (
                    commR_ref[s] + partial_ref[pl.ds(my * ROWS, HALF), :]
                )
                out_ref[pl.ds(HALF, HALF), :] = (
                    commL_ref[s]
                    + partial_ref[pl.ds(my * ROWS + HALF, HALF), :]
                )

    return pl.pallas_call(
        body,
        out_shape=jax.ShapeDtypeStruct((ROWS, D_OUT), jnp.float32),
        in_specs=[
            pl.BlockSpec(memory_space=pltpu.VMEM),
            pl.BlockSpec(memory_space=pltpu.VMEM),
            pl.BlockSpec(memory_space=pltpu.VMEM),
        ],
        out_specs=pl.BlockSpec(memory_space=pltpu.VMEM),
        scratch_shapes=[
            pltpu.VMEM((N_TOK, E_LOCAL), jnp.float32),
            pltpu.VMEM((N_TOK, D_OUT), jnp.float32),
            pltpu.VMEM((N_DEV - 2, HALF, D_OUT), jnp.float32),
            pltpu.VMEM((N_DEV - 2, HALF, D_OUT), jnp.float32),
            pltpu.VMEM((N_DEV - 1, HALF, D_OUT), jnp.float32),
            pltpu.VMEM((N_DEV - 1, HALF, D_OUT), jnp.float32),
            pltpu.SemaphoreType.DMA((N_DEV - 1,)),
            pltpu.SemaphoreType.DMA((N_DEV - 1,)),
            pltpu.SemaphoreType.DMA((N_DEV - 1,)),
            pltpu.SemaphoreType.DMA((N_DEV - 1,)),
        ],
        compiler_params=pltpu.CompilerParams(collective_id=0),
    )(x, route_idx, expert_W)


# device time: 30672 ns/iter; 1.7647x vs baseline; 1.1520x over previous
import jax
import jax.numpy as jnp
from jax import lax
from jax.experimental import pallas as pl
from jax.experimental.pallas import tpu as pltpu

N_DEV = 4
N_TOK = 1024
D_IN = 512
D_OUT = 1024
E_LOCAL = 4
CAP = 51
ROWS = N_TOK // N_DEV
HALF = ROWS // 2
N_SEG = 2
SEG = HALF // N_SEG
N_STEP = N_DEV - 1


def kernel(x, router_W, route_idx, expert_W):
    del router_W

    def body(x_ref, ridx_ref, ew_ref, out_ref,
             keep_ref, partial_ref, accR_ref, accL_ref,
             commR_ref, commL_ref, sendR, recvR, sendL, recvL):
        my = lax.axis_index("i")
        left = (my - 1) % N_DEV
        right = (my + 1) % N_DEV

        ridx = ridx_ref[:, :]
        local_ids = my * E_LOCAL + lax.broadcasted_iota(
            jnp.int32, (1, E_LOCAL), 1
        )
        onehot = (ridx == local_ids).astype(jnp.float32)
        cum = onehot
        sh = 1
        while sh < N_TOK:
            cum = cum + jnp.concatenate(
                [jnp.zeros((sh, E_LOCAL), jnp.float32), cum[:-sh]], axis=0
            )
            sh *= 2
        keep_ref[:, :] = onehot * (cum <= CAP).astype(jnp.float32)

        def compute_half(c, h):
            off = c * ROWS + h * HALF
            xc = x_ref[pl.ds(off, HALF), :]
            kc = keep_ref[pl.ds(off, HALF), :]
            acc = jnp.dot(xc * kc[:, 0:1], ew_ref[0],
                          preferred_element_type=jnp.float32)
            for j in range(1, E_LOCAL):
                acc = acc + jnp.dot(xc * kc[:, j:j + 1], ew_ref[j],
                                    preferred_element_type=jnp.float32)
            partial_ref[pl.ds(off, HALF), :] = acc

        cR0 = (my - 1) % N_DEV
        cL0 = (my + 1) % N_DEV

        compute_half(cR0, 0)
        compute_half(cL0, 1)

        barrier = pltpu.get_barrier_semaphore()
        for nbr in [left, right]:
            pl.semaphore_signal(
                barrier, inc=1,
                device_id=(nbr,), device_id_type=pl.DeviceIdType.MESH,
            )
        pl.semaphore_wait(barrier, 2)

        def make(s, g, ring):
            if ring == "R":
                comm, acc, snd, rcv, tgt, hoff = (
                    commR_ref, accR_ref, sendR, recvR, right, 0)
                c_send = (my - s - 1) % N_DEV
            else:
                comm, acc, snd, rcv, tgt, hoff = (
                    commL_ref, accL_ref, sendL, recvL, left, HALF)
                c_send = (my + s + 1) % N_DEV
            if s == 0:
                src = partial_ref.at[
                    pl.ds(c_send * ROWS + hoff + g * SEG, SEG), :]
            else:
                src = acc.at[s - 1, g]
            return pltpu.make_async_remote_copy(
                src_ref=src, dst_ref=comm.at[s, g],
                send_sem=snd.at[s, g], recv_sem=rcv.at[s, g],
                device_id=(tgt,), device_id_type=pl.DeviceIdType.MESH,
            )

        def add_recv(s, g, ring):
            if ring == "R":
                comm, acc, hoff = commR_ref, accR_ref, 0
                c_recv = (my - s - 2) % N_DEV
            else:
                comm, acc, hoff = commL_ref, accL_ref, HALF
                c_recv = (my + s + 2) % N_DEV
            mine = partial_ref[pl.ds(c_recv * ROWS + hoff + g * SEG, SEG), :]
            if s < N_STEP - 1:
                acc[s, g] = comm[s, g] + mine
            else:
                out_ref[pl.ds(hoff + g * SEG, SEG), :] = comm[s, g] + mine

        live = {}

        for g in range(N_SEG):
            for ring in ("R", "L"):
                r = make(0, g, ring)
                r.start()
                live[(0, g, ring)] = r

        flight_compute = {
            0: [((my + 2) % N_DEV, 0), ((my + 2) % N_DEV, 1)],
            1: [((my + 1) % N_DEV, 0), ((my - 1) % N_DEV, 1)],
            2: [(my, 0), (my, 1)],
        }
        for s in range(N_STEP):
            for c, h in flight_compute[s]:
                compute_half(c, h)
            for g in range(N_SEG):
                for ring in ("R", "L"):
                    live[(s, g, ring)].wait()
                    add_recv(s, g, ring)
                    if s + 1 < N_STEP:
                        r = make(s + 1, g, ring)
                        r.start()
                        live[(s + 1, g, ring)] = r

    return pl.pallas_call(
        body,
        out_shape=jax.ShapeDtypeStruct((ROWS, D_OUT), jnp.float32),
        in_specs=[
            pl.BlockSpec(memory_space=pltpu.VMEM),
            pl.BlockSpec(memory_space=pltpu.VMEM),
            pl.BlockSpec(memory_space=pltpu.VMEM),
        ],
        out_specs=pl.BlockSpec(memory_space=pltpu.VMEM),
        scratch_shapes=[
            pltpu.VMEM((N_TOK, E_LOCAL), jnp.float32),
            pltpu.VMEM((N_TOK, D_OUT), jnp.float32),
            pltpu.VMEM((N_STEP - 1, N_SEG, SEG, D_OUT), jnp.float32),
            pltpu.VMEM((N_STEP - 1, N_SEG, SEG, D_OUT), jnp.float32),
            pltpu.VMEM((N_STEP, N_SEG, SEG, D_OUT), jnp.float32),
            pltpu.VMEM((N_STEP, N_SEG, SEG, D_OUT), jnp.float32),
            pltpu.SemaphoreType.DMA((N_STEP, N_SEG)),
            pltpu.SemaphoreType.DMA((N_STEP, N_SEG)),
            pltpu.SemaphoreType.DMA((N_STEP, N_SEG)),
            pltpu.SemaphoreType.DMA((N_STEP, N_SEG)),
        ],
        compiler_params=pltpu.CompilerParams(collective_id=0),
    )(x, route_idx, expert_W)


# device time: 30238 ns/iter; 1.7901x vs baseline; 1.0144x over previous
import jax
import jax.numpy as jnp
from jax import lax
from jax.experimental import pallas as pl
from jax.experimental.pallas import tpu as pltpu

N_DEV = 4
N_TOK = 1024
D_IN = 512
D_OUT = 1024
N_EXP = 16
E_LOCAL = 4
CAP = 51
PAD_E = 56
BLK = E_LOCAL * PAD_E
HALF = BLK // 2
N_SEG = 2
SEG = HALF // N_SEG
ROWS = N_TOK // N_DEV
N_STEP = N_DEV - 1


def kernel(x, router_W, route_idx, expert_W):
    del router_W

    def body(x_ref, ridx_ref, ew_ref, out_ref,
             route_scr, y_ref, commR_ref, commL_ref,
             sendR, recvR, sendL, recvL):
        my = lax.axis_index("i")
        left = (my - 1) % N_DEV
        right = (my + 1) % N_DEV

        ridx = ridx_ref[:, :]
        eids = lax.broadcasted_iota(jnp.int32, (1, N_EXP), 1)
        onehot = (ridx == eids).astype(jnp.float32)
        cum = onehot
        sh = 1
        while sh < N_TOK:
            cum = cum + jnp.concatenate(
                [jnp.zeros((sh, N_EXP), jnp.float32), cum[:-sh]], axis=0
            )
            sh *= 2
        rank = jnp.sum(onehot * cum, axis=1, keepdims=True)
        kept = jnp.sum(onehot * (cum <= CAP).astype(jnp.float32),
                       axis=1, keepdims=True)
        eloc = (ridx % E_LOCAL).astype(jnp.float32)
        chip = (ridx // E_LOCAL).astype(jnp.float32)
        slot = jnp.where(kept > 0.0, eloc * PAD_E + rank - 1.0, -1.0)
        route_scr[:, 0:1] = slot
        route_scr[:, 1:2] = chip

        slot_col = route_scr[:, 0:1]
        chip_col = route_scr[:, 1:2]
        myf = my.astype(jnp.float32)
        slot_lane = lax.broadcasted_iota(jnp.int32, (1, BLK), 1).astype(jnp.float32)
        g_t = ((slot_col == slot_lane).astype(jnp.float32)
               * (chip_col == myf).astype(jnp.float32))
        x_c = lax.dot_general(
            g_t, x_ref[:, :], (((0,), (0,)), ((), ())),
            preferred_element_type=jnp.float32,
        )
        for j in range(E_LOCAL):
            y_ref[pl.ds(j * PAD_E, PAD_E), :] = jnp.dot(
                x_c[j * PAD_E:(j + 1) * PAD_E, :], ew_ref[j],
                preferred_element_type=jnp.float32,
            )

        barrier = pltpu.get_barrier_semaphore()
        for nbr in [left, right]:
            pl.semaphore_signal(
                barrier, inc=1,
                device_id=(nbr,), device_id_type=pl.DeviceIdType.MESH,
            )
        pl.semaphore_wait(barrier, 2)

        def make(h, g, ring):
            if ring == "R":
                comm, snd, rcv, tgt, hoff = commR_ref, sendR, recvR, right, 0
            else:
                comm, snd, rcv, tgt, hoff = commL_ref, sendL, recvL, left, HALF
            if h == 0:
                src = y_ref.at[pl.ds(hoff + g * SEG, SEG), :]
            else:
                src = comm.at[h - 1, g]
            return pltpu.make_async_remote_copy(
                src_ref=src, dst_ref=comm.at[h, g],
                send_sem=snd.at[h, g], recv_sem=rcv.at[h, g],
                device_id=(tgt,), device_id_type=pl.DeviceIdType.MESH,
            )

        live = {}
        for g in range(N_SEG):
            for ring in ("R", "L"):
                r = make(0, g, ring)
                r.start()
                live[(0, g, ring)] = r

        slot_q = route_scr[pl.ds(my * ROWS, ROWS), 0:1]
        chip_q = route_scr[pl.ds(my * ROWS, ROWS), 1:2]

        def scatter_piece(o, base, block):
            rows = block.shape[0]
            lane = base + lax.broadcasted_iota(jnp.int32, (1, rows), 1).astype(jnp.float32)
            s_oh = ((slot_q == lane).astype(jnp.float32)
                    * (chip_q == o.astype(jnp.float32)).astype(jnp.float32))
            out_ref[:, :] = out_ref[:, :] + jnp.dot(
                s_oh, block, preferred_element_type=jnp.float32
            )

        lane_all = lax.broadcasted_iota(jnp.int32, (1, BLK), 1).astype(jnp.float32)
        s_own = ((slot_q == lane_all).astype(jnp.float32)
                 * (chip_q == myf).astype(jnp.float32))
        out_ref[:, :] = jnp.dot(s_own, y_ref[:, :],
                                preferred_element_type=jnp.float32)

        for h in range(N_STEP):
            for g in range(N_SEG):
                for ring in ("R", "L"):
                    live[(h, g, ring)].wait()
                    if h + 1 < N_STEP:
                        r = make(h + 1, g, ring)
                        r.start()
                        live[(h + 1, g, ring)] = r
                    if ring == "R":
                        o = (my - h - 1) % N_DEV
                        scatter_piece(o, float(g * SEG), commR_ref[h, g])
                    else:
                        o = (my + h + 1) % N_DEV
                        scatter_piece(o, float(HALF + g * SEG),
                                      commL_ref[h, g])

    return pl.pallas_call(
        body,
        out_shape=jax.ShapeDtypeStruct((ROWS, D_OUT), jnp.float32),
        in_specs=[
            pl.BlockSpec(memory_space=pltpu.VMEM),
            pl.BlockSpec(memory_space=pltpu.VMEM),
            pl.BlockSpec(memory_space=pltpu.VMEM),
        ],
        out_specs=pl.BlockSpec(memory_space=pltpu.VMEM),
        scratch_shapes=[
            pltpu.VMEM((N_TOK, 2), jnp.float32),
            pltpu.VMEM((BLK, D_OUT), jnp.float32),
            pltpu.VMEM((N_STEP, N_SEG, SEG, D_OUT), jnp.float32),
            pltpu.VMEM((N_STEP, N_SEG, SEG, D_OUT), jnp.float32),
            pltpu.SemaphoreType.DMA((N_STEP, N_SEG)),
            pltpu.SemaphoreType.DMA((N_STEP, N_SEG)),
            pltpu.SemaphoreType.DMA((N_STEP, N_SEG)),
            pltpu.SemaphoreType.DMA((N_STEP, N_SEG)),
        ],
        compiler_params=pltpu.CompilerParams(collective_id=0),
    )(x, route_idx, expert_W)


# device time: 29314 ns/iter; 1.8465x vs baseline; 1.0315x over previous
import jax
import jax.numpy as jnp
from jax import lax
from jax.experimental import pallas as pl
from jax.experimental.pallas import tpu as pltpu

N_DEV = 4
N_TOK = 1024
D_IN = 512
D_OUT = 1024
N_EXP = 16
E_LOCAL = 4
CAP = 51
PAD_E = 56
BLK = E_LOCAL * PAD_E
GATH = N_DEV * BLK
HALF = BLK // 2
N_SEG = 2
SEG = HALF // N_SEG
ROWS = N_TOK // N_DEV
N_STEP = N_DEV - 1


def kernel(x, router_W, route_idx, expert_W):
    del router_W

    def body(x_ref, ridx_ref, ew_ref, out_ref,
             gslot_ref, yall_ref, s_ref,
             sendR, recvR, sendL, recvL):
        my = lax.axis_index("i")
        left = (my - 1) % N_DEV
        right = (my + 1) % N_DEV

        ridx = ridx_ref[:, :]
        eids = lax.broadcasted_iota(jnp.int32, (1, N_EXP), 1)
        onehot = (ridx == eids).astype(jnp.float32)
        cum = onehot
        sh = 1
        while sh < N_TOK:
            cum = cum + jnp.concatenate(
                [jnp.zeros((sh, N_EXP), jnp.float32), cum[:-sh]], axis=0
            )
            sh *= 2
        rank = jnp.sum(onehot * cum, axis=1, keepdims=True)
        kept = jnp.sum(onehot * (cum <= CAP).astype(jnp.float32),
                       axis=1, keepdims=True)
        eloc = (ridx % E_LOCAL).astype(jnp.float32)
        chip = (ridx // E_LOCAL).astype(jnp.float32)
        gslot = jnp.where(
            kept > 0.0, chip * BLK + eloc * PAD_E + rank - 1.0, -1.0
        )
        gslot_ref[:, :] = gslot

        my_base = my * BLK
        slot_col = gslot_ref[:, :] - my_base.astype(jnp.float32)
        slot_lane = lax.broadcasted_iota(
            jnp.int32, (1, BLK), 1).astype(jnp.float32)
        g_t = (slot_col == slot_lane).astype(jnp.float32)
        x_c = lax.dot_general(
            g_t, x_ref[:, :], (((0,), (0,)), ((), ())),
            preferred_element_type=jnp.float32,
        )
        for j in range(E_LOCAL):
            yall_ref[pl.ds(my_base + j * PAD_E, PAD_E), :] = jnp.dot(
                x_c[j * PAD_E:(j + 1) * PAD_E, :], ew_ref[j],
                preferred_element_type=jnp.float32,
            )

        barrier = pltpu.get_barrier_semaphore()
        for nbr in [left, right]:
            pl.semaphore_signal(
                barrier, inc=1,
                device_id=(nbr,), device_id_type=pl.DeviceIdType.MESH,
            )
        pl.semaphore_wait(barrier, 2)

        def make(h, g, ring):
            if ring == "R":
                snd, rcv, tgt, hoff = sendR, recvR, right, 0
                origin = (my - h) % N_DEV
            else:
                snd, rcv, tgt, hoff = sendL, recvL, left, HALF
                origin = (my + h) % N_DEV
            sl = pl.ds(origin * BLK + hoff + g * SEG, SEG)
            return pltpu.make_async_remote_copy(
                src_ref=yall_ref.at[sl, :], dst_ref=yall_ref.at[sl, :],
                send_sem=snd.at[h, g], recv_sem=rcv.at[h, g],
                device_id=(tgt,), device_id_type=pl.DeviceIdType.MESH,
            )

        live = {}
        for g in range(N_SEG):
            for ring in ("R", "L"):
                r = make(0, g, ring)
                r.start()
                live[(0, g, ring)] = r

        gslot_q = gslot_ref[pl.ds(my * ROWS, ROWS), :]
        lane_all = lax.broadcasted_iota(
            jnp.int32, (1, GATH), 1).astype(jnp.float32)
        s_ref[:, :] = (gslot_q == lane_all).astype(jnp.float32)

        for h in range(N_STEP):
            for g in range(N_SEG):
                for ring in ("R", "L"):
                    live[(h, g, ring)].wait()
                    if h + 1 < N_STEP:
                        r = make(h + 1, g, ring)
                        r.start()
                        live[(h + 1, g, ring)] = r

        out_ref[:, :] = jnp.dot(s_ref[:, :], yall_ref[:, :],
                                preferred_element_type=jnp.float32)

    return pl.pallas_call(
        body,
        out_shape=jax.ShapeDtypeStruct((ROWS, D_OUT), jnp.float32),
        in_specs=[
            pl.BlockSpec(memory_space=pltpu.VMEM),
            pl.BlockSpec(memory_space=pltpu.VMEM),
            pl.BlockSpec(memory_space=pltpu.VMEM),
        ],
        out_specs=pl.BlockSpec(memory_space=pltpu.VMEM),
        scratch_shapes=[
            pltpu.VMEM((N_TOK, 1), jnp.float32),
            pltpu.VMEM((GATH, D_OUT), jnp.float32),
            pltpu.VMEM((ROWS, GATH), jnp.float32),
            pltpu.SemaphoreType.DMA((N_STEP, N_SEG)),
            pltpu.SemaphoreType.DMA((N_STEP, N_SEG)),
            pltpu.SemaphoreType.DMA((N_STEP, N_SEG)),
            pltpu.SemaphoreType.DMA((N_STEP, N_SEG)),
        ],
        compiler_params=pltpu.CompilerParams(collective_id=0),
    )(x, route_idx, expert_W)


# device time: 24203 ns/iter; 2.2364x vs baseline; 1.2112x over previous
import jax
import jax.numpy as jnp
from jax import lax
from jax.experimental import pallas as pl
from jax.experimental.pallas import tpu as pltpu

N_DEV = 4
N_TOK = 1024
D_IN = 512
D_OUT = 1024
N_EXP = 16
E_LOCAL = 4
CAP = 51
PAD_E = 64
BLK = E_LOCAL * PAD_E
GATH = N_DEV * BLK
HALF = BLK // 2
N_SEG = 2
SEG = HALF // N_SEG
ROWS = N_TOK // N_DEV
N_STEP = N_DEV - 1


def kernel(x, router_W, route_idx, expert_W):
    del router_W

    def body(x_ref, ridx_ref, ew_ref, out_ref,
             gslot_ref, yall_ref, s_ref,
             sendR, recvR, sendL, recvL):
        my = lax.axis_index("i")
        left = (my - 1) % N_DEV
        right = (my + 1) % N_DEV

        barrier = pltpu.get_barrier_semaphore()
        for nbr in [left, right]:
            pl.semaphore_signal(
                barrier, inc=1,
                device_id=(nbr,), device_id_type=pl.DeviceIdType.MESH,
            )
        pl.semaphore_wait(barrier, 2)

        ridx = ridx_ref[:, :]
        eids = lax.broadcasted_iota(jnp.int32, (1, N_EXP), 1)
        onehot = (ridx == eids).astype(jnp.float32)
        cum = onehot
        sh = 1
        while sh < N_TOK:
            cum = cum + jnp.concatenate(
                [jnp.zeros((sh, N_EXP), jnp.float32), cum[:-sh]], axis=0
            )
            sh *= 2
        rank = jnp.sum(onehot * cum, axis=1, keepdims=True)
        kept = jnp.sum(onehot * (cum <= CAP).astype(jnp.float32),
                       axis=1, keepdims=True)
        eloc = (ridx % E_LOCAL).astype(jnp.float32)
        chip = (ridx // E_LOCAL).astype(jnp.float32)
        gslot = jnp.where(
            kept > 0.0, chip * BLK + eloc * PAD_E + rank - 1.0, -1.0
        )
        gslot_ref[:, :] = gslot

        def make(h, g, ring):
            if ring == "R":
                snd, rcv, tgt, hoff = sendR, recvR, right, 0
                origin = (my - h) % N_DEV
            else:
                snd, rcv, tgt, hoff = sendL, recvL, left, HALF
                origin = (my + h) % N_DEV
            sl = pl.ds(origin * BLK + hoff + g * SEG, SEG)
            return pltpu.make_async_remote_copy(
                src_ref=yall_ref.at[sl, :], dst_ref=yall_ref.at[sl, :],
                send_sem=snd.at[h, g], recv_sem=rcv.at[h, g],
                device_id=(tgt,), device_id_type=pl.DeviceIdType.MESH,
            )

        my_base = my * BLK
        slot_col = gslot_ref[:, :] - my_base.astype(jnp.float32)
        slot_lane = lax.broadcasted_iota(
            jnp.int32, (1, BLK), 1).astype(jnp.float32)
        g_t = (slot_col == slot_lane).astype(jnp.float32)
        x_c = lax.dot_general(
            g_t, x_ref[:, :], (((0,), (0,)), ((), ())),
            preferred_element_type=jnp.float32,
        )
        live = {}
        hop0 = {0: (0, "R"), 1: (1, "R"), 2: (0, "L"), 3: (1, "L")}
        for j in range(E_LOCAL):
            yall_ref[pl.ds(my_base + j * PAD_E, PAD_E), :] = jnp.dot(
                x_c[j * PAD_E:(j + 1) * PAD_E, :], ew_ref[j],
                preferred_element_type=jnp.float32,
            ).astype(jnp.bfloat16)
            g, ring = hop0[j]
            r = make(0, g, ring)
            r.start()
            live[(0, g, ring)] = r

        gslot_q = gslot_ref[pl.ds(my * ROWS, ROWS), :]
        lane_all = lax.broadcasted_iota(
            jnp.int32, (1, GATH), 1).astype(jnp.float32)
        s_ref[:, :] = (gslot_q == lane_all).astype(jnp.bfloat16)

        for h in range(N_STEP):
            for g in range(N_SEG):
                for ring in ("R", "L"):
                    live[(h, g, ring)].wait()
                    if h + 1 < N_STEP:
                        r = make(h + 1, g, ring)
                        r.start()
                        live[(h + 1, g, ring)] = r

        out_ref[:, :] = jnp.dot(s_ref[:, :], yall_ref[:, :],
                                preferred_element_type=jnp.float32)

    return pl.pallas_call(
        body,
        out_shape=jax.ShapeDtypeStruct((ROWS, D_OUT), jnp.float32),
        in_specs=[
            pl.BlockSpec(memory_space=pltpu.VMEM),
            pl.BlockSpec(memory_space=pltpu.VMEM),
            pl.BlockSpec(memory_space=pltpu.VMEM),
        ],
        out_specs=pl.BlockSpec(memory_space=pltpu.VMEM),
        scratch_shapes=[
            pltpu.VMEM((N_TOK, 1), jnp.float32),
            pltpu.VMEM((GATH, D_OUT), jnp.bfloat16),
            pltpu.VMEM((ROWS, GATH), jnp.bfloat16),
            pltpu.SemaphoreType.DMA((N_STEP, N_SEG)),
            pltpu.SemaphoreType.DMA((N_STEP, N_SEG)),
            pltpu.SemaphoreType.DMA((N_STEP, N_SEG)),
            pltpu.SemaphoreType.DMA((N_STEP, N_SEG)),
        ],
        compiler_params=pltpu.CompilerParams(collective_id=0),
    )(x, route_idx, expert_W)
